# baseline (speedup 1.0000x reference)
import functools

import numpy as np
import jax
import jax.numpy as jnp
from jax import lax
from jax.experimental import pallas as pl
from jax.experimental.pallas import tpu as pltpu
from jax.experimental.pallas import tpu_sc as plsc

_N = 5000
_LANES = 16
_NPAD = 5008
_NCHUNK = _NPAD // _LANES
_MIN_LEN = 6.0
_SUPPRESS_LT = float(np.nextafter(np.float32(4.0), np.float32(0.0)))
_G = 64
_CAP = 4
_GSLOTS = _G * _G * _CAP
_GPAD = _GSLOTS + _LANES
_OV = 256
_OVPAD = _OV + _LANES
_SENT = 1e18
_FULL = 8.9e17


def _nms_kernel_body(xs_hbm, ys_hbm, sent_hbm, keep_hbm,
                     xs_v, ys_v, alive_v, gx_v, gy_v, ox_v, oy_v, cnt_v, ovs):
    lane_iota = lax.iota(jnp.int32, _LANES)
    ones = jnp.broadcast_to(jnp.float32(1.0), (_LANES,))
    zeros = jnp.broadcast_to(jnp.float32(0.0), (_LANES,))

    @pl.when((lax.axis_index("c") == 0) & (lax.axis_index("s") == 0))
    def _work():
        pltpu.sync_copy(xs_hbm, xs_v)
        pltpu.sync_copy(ys_hbm, ys_v)
        pltpu.sync_copy(sent_hbm, gx_v)
        pltpu.sync_copy(sent_hbm, gy_v)
        pltpu.sync_copy(sent_hbm.at[pl.ds(0, _OVPAD)], ox_v)
        pltpu.sync_copy(sent_hbm.at[pl.ds(0, _OVPAD)], oy_v)
        ovs[0] = jnp.int32(0)

        def _zc(c, carry):
            cnt_v[pl.ds(c * _LANES, _LANES)] = lane_iota * 0
            return carry

        lax.fori_loop(0, (_G * _G) // _LANES, _zc, 0)

        def _outer(cj, nselv):
            base = cj * _LANES
            sl = pl.ds(base, _LANES)
            xv = xs_v[sl]
            yv = ys_v[sl]

            fxv = xv * 0.5
            fyv = yv * 0.5
            txv = fxv.astype(jnp.int32)
            tyv = fyv.astype(jnp.int32)
            ixv = txv - jnp.where(fxv < txv.astype(jnp.float32), 1, 0)
            iyv = tyv - jnp.where(fyv < tyv.astype(jnp.float32), 1, 0)
            hitv = zeros
            homebase = None
            for dyy in (-1, 0, 1):
                rowv = ((iyv + dyy) & (_G - 1)) << 6
                for dxx in (-1, 0, 1):
                    basev = (rowv + ((ixv + dxx) & (_G - 1))) << 2
                    if dxx == 0 and dyy == 0:
                        homebase = basev
                    for s in range(_CAP):
                        idxv = basev + s
                        gxs = plsc.load_gather(gx_v, [idxv])
                        gys = plsc.load_gather(gy_v, [idxv])
                        ddx = gxs - xv
                        ddy = gys - yv
                        d2 = ddx * ddx + ddy * ddy
                        hitv = jnp.where(d2 < _SUPPRESS_LT, ones, hitv)

            ovcnt = ovs[0]

            def _ovchunk(c, hv):
                o = pl.ds(c * _LANES, _LANES)
                oxc = ox_v[o]
                oyc = oy_v[o]
                for e in range(_LANES):
                    dxe = xv - oxc[e]
                    dye = yv - oyc[e]
                    d2e = dxe * dxe + dye * dye
                    hv = jnp.where(d2e < _SUPPRESS_LT, ones, hv)
                return hv

            hitv = lax.fori_loop(0, (ovcnt + 15) >> 4, _ovchunk, hitv)

            validv = jnp.where((base + lane_iota) < _N, ones, zeros)
            av = (ones - hitv) * validv
            for l in range(_LANES):
                dx = xv - xv[l]
                dy = yv - yv[l]
                d2 = dx * dx + dy * dy
                gate = jnp.where(lane_iota > l, av[l], jnp.float32(0.0))
                hitf = jnp.where(d2 < _SUPPRESS_LT, gate, zeros)
                av = av * (ones - hitf)
            alive_v[sl] = av

            homecell = homebase >> 2
            cntv = plsc.load_gather(cnt_v, [homecell])
            izeros = lane_iota * 0
            iones = izeros + 1
            dupoff = izeros
            for o in range(1, _LANES):
                shl = (lane_iota - o) & (_LANES - 1)
                hb_b = homecell.at[shl].get(mode="promise_in_bounds")
                av_b = av.at[shl].get(mode="promise_in_bounds")
                sb = jnp.where(homecell == hb_b, av_b, jnp.float32(0.0))
                sb = jnp.where(lane_iota >= o, sb, jnp.float32(0.0))
                dupoff = dupoff + jnp.where(sb > 0.5, iones, izeros)
            slotv = cntv + dupoff
            okf = jnp.where(slotv < _CAP, av, zeros)
            plsc.store_scatter(gx_v, [homebase + slotv], xv, mask=okf > 0.5)
            plsc.store_scatter(gy_v, [homebase + slotv], yv, mask=okf > 0.5)
            plsc.addupdate_scatter(cnt_v, [homecell], iones, mask=av > 0.5)

            ovff = av - okf
            ovfs = ovff[0]
            for l in range(1, _LANES):
                ovfs = ovfs + ovff[l]

            @pl.when(ovfs > 0.5)
            def _ov_all():
                for l in range(_LANES):
                    @pl.when(ovff[l] > 0.5)
                    def _ov_ins(l=l):
                        ovc = ovs[0]

                        @pl.when(ovc < _OV)
                        def _ov_ins2():
                            ob = (ovc >> 4) << 4
                            olane = ovc - ob
                            osl = pl.ds(ob, _LANES)
                            ox_v[osl] = jnp.where(lane_iota == olane, xv[l],
                                                  ox_v[osl])
                            oy_v[osl] = jnp.where(lane_iota == olane, yv[l],
                                                  oy_v[osl])

                        ovs[0] = ovc + 1

            return nselv + av

        nselv = lax.fori_loop(0, _NCHUNK, _outer, zeros)
        nsel = nselv[0]
        for l in range(1, _LANES):
            nsel = nsel + nselv[l]

        need = jnp.maximum(jnp.float32(_MIN_LEN) - nsel, 0.0)

        @pl.when(need > 0.5)
        def _backfill():
            def _bf(c, run):
                base = c * _LANES
                sl = pl.ds(base, _LANES)
                av = alive_v[sl]
                newav = av
                for l in range(_LANES):
                    valid = (base + l) < _N
                    notk = valid & (av[l] < 0.5)
                    takef = jnp.where(notk & (run < need),
                                      jnp.float32(1.0), jnp.float32(0.0))
                    newav = newav + jnp.where(lane_iota == l, takef,
                                              jnp.float32(0.0))
                    run = run + jnp.where(notk, jnp.float32(1.0),
                                          jnp.float32(0.0))
                alive_v[sl] = newav
                return run

            lax.fori_loop(0, _NCHUNK, _bf, jnp.float32(0.0))

        pltpu.sync_copy(alive_v, keep_hbm)


@jax.jit
def _nms_keep_mask(xs_pad, ys_pad, sent):
    fn = pl.kernel(
        _nms_kernel_body,
        out_type=jax.ShapeDtypeStruct((_NPAD,), jnp.float32),
        mesh=plsc.VectorSubcoreMesh(core_axis_name="c", subcore_axis_name="s"),
        compiler_params=pltpu.CompilerParams(needs_layout_passes=False),
        scratch_types=[
            pltpu.VMEM((_NPAD,), jnp.float32),
            pltpu.VMEM((_NPAD,), jnp.float32),
            pltpu.VMEM((_NPAD,), jnp.float32),
            pltpu.VMEM((_GPAD,), jnp.float32),
            pltpu.VMEM((_GPAD,), jnp.float32),
            pltpu.VMEM((_OVPAD,), jnp.float32),
            pltpu.VMEM((_OVPAD,), jnp.float32),
            pltpu.VMEM((_G * _G,), jnp.int32),
            pltpu.SMEM((1,), jnp.int32),
        ],
    )
    return fn(xs_pad, ys_pad, sent)


def kernel(xys, logits):
    order = jnp.argsort(-logits)
    xys_sorted = jnp.take(xys, order, axis=0)
    pad = jnp.full((_NPAD - _N,), 1e9, dtype=jnp.float32)
    xs_pad = jnp.concatenate([xys_sorted[:, 0], pad])
    ys_pad = jnp.concatenate([xys_sorted[:, 1], pad])
    sent = jnp.full((_GPAD,), _SENT, dtype=jnp.float32)
    keep_f = _nms_keep_mask(xs_pad, ys_pad, sent)[:_N]
    keep_final = keep_f > 0.5
    selected_idcs = jnp.where(keep_final, order, -1)
    selected_xys = xys_sorted * keep_f[:, None]
    return selected_idcs, selected_xys, keep_final


# speedup vs baseline: 1.0782x; 1.0782x over previous
import functools

import numpy as np
import jax
import jax.numpy as jnp
from jax import lax
from jax.experimental import pallas as pl
from jax.experimental.pallas import tpu as pltpu
from jax.experimental.pallas import tpu_sc as plsc

_N = 5000
_LANES = 16
_NPAD = 5008
_NCHUNK = _NPAD // _LANES
_MIN_LEN = 6.0
_SUPPRESS_LT = float(np.nextafter(np.float32(4.0), np.float32(0.0)))
_G = 64
_CAP = 4
_GSLOTS = _G * _G * _CAP
_GPAD = _GSLOTS + _LANES
_OV = 256
_OVPAD = _OV + _LANES
_SENT = 1e18
_FULL = 8.9e17


def _nms_kernel_body(xs_hbm, ys_hbm, sent_hbm, keep_hbm,
                     xs_v, ys_v, alive_v, gx_v, gy_v, ox_v, oy_v, cnt_v, ovs):
    lane_iota = lax.iota(jnp.int32, _LANES)
    ones = jnp.broadcast_to(jnp.float32(1.0), (_LANES,))
    zeros = jnp.broadcast_to(jnp.float32(0.0), (_LANES,))

    @pl.when((lax.axis_index("c") == 0) & (lax.axis_index("s") == 0))
    def _work():
        pltpu.sync_copy(xs_hbm, xs_v)
        pltpu.sync_copy(ys_hbm, ys_v)
        pltpu.sync_copy(sent_hbm, gx_v)
        pltpu.sync_copy(sent_hbm, gy_v)
        pltpu.sync_copy(sent_hbm.at[pl.ds(0, _OVPAD)], ox_v)
        pltpu.sync_copy(sent_hbm.at[pl.ds(0, _OVPAD)], oy_v)
        ovs[0] = jnp.int32(0)

        def _zc(c, carry):
            cnt_v[pl.ds(c * _LANES, _LANES)] = lane_iota * 0
            return carry

        lax.fori_loop(0, (_G * _G) // _LANES, _zc, 0)

        def _outer(cj, nselv):
            base = cj * _LANES
            sl = pl.ds(base, _LANES)
            xv = xs_v[sl]
            yv = ys_v[sl]

            fxv = xv * 0.5
            fyv = yv * 0.5
            txv = fxv.astype(jnp.int32)
            tyv = fyv.astype(jnp.int32)
            ixv = txv - jnp.where(fxv < txv.astype(jnp.float32), 1, 0)
            iyv = tyv - jnp.where(fyv < tyv.astype(jnp.float32), 1, 0)
            hitv = zeros
            homebase = None
            for dyy in (-1, 0, 1):
                rowv = ((iyv + dyy) & (_G - 1)) << 6
                for dxx in (-1, 0, 1):
                    basev = (rowv + ((ixv + dxx) & (_G - 1))) << 2
                    if dxx == 0 and dyy == 0:
                        homebase = basev
                    for s in range(_CAP):
                        idxv = basev + s
                        gxs = plsc.load_gather(gx_v, [idxv])
                        gys = plsc.load_gather(gy_v, [idxv])
                        ddx = gxs - xv
                        ddy = gys - yv
                        d2 = ddx * ddx + ddy * ddy
                        hitv = jnp.where(d2 < _SUPPRESS_LT, ones, hitv)

            ovcnt = ovs[0]

            def _ovchunk(c, hv):
                o = pl.ds(c * _LANES, _LANES)
                oxc = ox_v[o]
                oyc = oy_v[o]
                for e in range(_LANES):
                    dxe = xv - oxc[e]
                    dye = yv - oyc[e]
                    d2e = dxe * dxe + dye * dye
                    hv = jnp.where(d2e < _SUPPRESS_LT, ones, hv)
                return hv

            hitv = lax.fori_loop(0, (ovcnt + 15) >> 4, _ovchunk, hitv)

            validv = jnp.where((base + lane_iota) < _N, ones, zeros)
            avp = (ones - hitv) * validv
            confv = zeros
            for o in range(1, 9):
                sh = (lane_iota + o) & (_LANES - 1)
                xr = xv.at[sh].get(mode="promise_in_bounds")
                yr = yv.at[sh].get(mode="promise_in_bounds")
                dxr = xv - xr
                dyr = yv - yr
                d2r = dxr * dxr + dyr * dyr
                confv = jnp.where(d2r < _SUPPRESS_LT, ones, confv)
            for s in (8, 4, 2, 1):
                confv = jnp.maximum(
                    confv, confv.at[lane_iota ^ s].get(mode="promise_in_bounds"))
            alive_v[sl] = avp

            @pl.when(confv[0] > 0.5)
            def _resolve():
                av = avp
                for l in range(_LANES):
                    dx = xv - xv[l]
                    dy = yv - yv[l]
                    d2 = dx * dx + dy * dy
                    gate = jnp.where(lane_iota > l, av[l], jnp.float32(0.0))
                    hitf = jnp.where(d2 < _SUPPRESS_LT, gate, zeros)
                    av = av * (ones - hitf)
                alive_v[sl] = av

            av = alive_v[sl]

            homecell = homebase >> 2
            cntv = plsc.load_gather(cnt_v, [homecell])
            izeros = lane_iota * 0
            iones = izeros + 1
            dupoff = izeros
            for o in range(1, _LANES):
                shl = (lane_iota - o) & (_LANES - 1)
                hb_b = homecell.at[shl].get(mode="promise_in_bounds")
                av_b = av.at[shl].get(mode="promise_in_bounds")
                sb = jnp.where(homecell == hb_b, av_b, jnp.float32(0.0))
                sb = jnp.where(lane_iota >= o, sb, jnp.float32(0.0))
                dupoff = dupoff + jnp.where(sb > 0.5, iones, izeros)
            slotv = cntv + dupoff
            okf = jnp.where(slotv < _CAP, av, zeros)
            plsc.store_scatter(gx_v, [homebase + slotv], xv, mask=okf > 0.5)
            plsc.store_scatter(gy_v, [homebase + slotv], yv, mask=okf > 0.5)
            plsc.addupdate_scatter(cnt_v, [homecell], iones, mask=av > 0.5)

            ovff = av - okf
            ovfs = ovff[0]
            for l in range(1, _LANES):
                ovfs = ovfs + ovff[l]

            @pl.when(ovfs > 0.5)
            def _ov_all():
                for l in range(_LANES):
                    @pl.when(ovff[l] > 0.5)
                    def _ov_ins(l=l):
                        ovc = ovs[0]

                        @pl.when(ovc < _OV)
                        def _ov_ins2():
                            ob = (ovc >> 4) << 4
                            olane = ovc - ob
                            osl = pl.ds(ob, _LANES)
                            ox_v[osl] = jnp.where(lane_iota == olane, xv[l],
                                                  ox_v[osl])
                            oy_v[osl] = jnp.where(lane_iota == olane, yv[l],
                                                  oy_v[osl])

                        ovs[0] = ovc + 1

            return nselv + av

        nselv = lax.fori_loop(0, _NCHUNK, _outer, zeros)
        nsel = nselv[0]
        for l in range(1, _LANES):
            nsel = nsel + nselv[l]

        need = jnp.maximum(jnp.float32(_MIN_LEN) - nsel, 0.0)

        @pl.when(need > 0.5)
        def _backfill():
            def _bf(c, run):
                base = c * _LANES
                sl = pl.ds(base, _LANES)
                av = alive_v[sl]
                newav = av
                for l in range(_LANES):
                    valid = (base + l) < _N
                    notk = valid & (av[l] < 0.5)
                    takef = jnp.where(notk & (run < need),
                                      jnp.float32(1.0), jnp.float32(0.0))
                    newav = newav + jnp.where(lane_iota == l, takef,
                                              jnp.float32(0.0))
                    run = run + jnp.where(notk, jnp.float32(1.0),
                                          jnp.float32(0.0))
                alive_v[sl] = newav
                return run

            lax.fori_loop(0, _NCHUNK, _bf, jnp.float32(0.0))

        pltpu.sync_copy(alive_v, keep_hbm)


@jax.jit
def _nms_keep_mask(xs_pad, ys_pad, sent):
    fn = pl.kernel(
        _nms_kernel_body,
        out_type=jax.ShapeDtypeStruct((_NPAD,), jnp.float32),
        mesh=plsc.VectorSubcoreMesh(core_axis_name="c", subcore_axis_name="s"),
        compiler_params=pltpu.CompilerParams(needs_layout_passes=False),
        scratch_types=[
            pltpu.VMEM((_NPAD,), jnp.float32),
            pltpu.VMEM((_NPAD,), jnp.float32),
            pltpu.VMEM((_NPAD,), jnp.float32),
            pltpu.VMEM((_GPAD,), jnp.float32),
            pltpu.VMEM((_GPAD,), jnp.float32),
            pltpu.VMEM((_OVPAD,), jnp.float32),
            pltpu.VMEM((_OVPAD,), jnp.float32),
            pltpu.VMEM((_G * _G,), jnp.int32),
            pltpu.SMEM((1,), jnp.int32),
        ],
    )
    return fn(xs_pad, ys_pad, sent)


def kernel(xys, logits):
    order = jnp.argsort(-logits)
    xys_sorted = jnp.take(xys, order, axis=0)
    pad = jnp.full((_NPAD - _N,), 1e9, dtype=jnp.float32)
    xs_pad = jnp.concatenate([xys_sorted[:, 0], pad])
    ys_pad = jnp.concatenate([xys_sorted[:, 1], pad])
    sent = jnp.full((_GPAD,), _SENT, dtype=jnp.float32)
    keep_f = _nms_keep_mask(xs_pad, ys_pad, sent)[:_N]
    keep_final = keep_f > 0.5
    selected_idcs = jnp.where(keep_final, order, -1)
    selected_xys = xys_sorted * keep_f[:, None]
    return selected_idcs, selected_xys, keep_final


# speedup vs baseline: 1.1922x; 1.1057x over previous
import functools

import numpy as np
import jax
import jax.numpy as jnp
from jax import lax
from jax.experimental import pallas as pl
from jax.experimental.pallas import tpu as pltpu
from jax.experimental.pallas import tpu_sc as plsc

_N = 5000
_LANES = 16
_NPAD = 5008
_NCHUNK = _NPAD // _LANES
_MIN_LEN = 6.0
_SUPPRESS_LT = float(np.nextafter(np.float32(4.0), np.float32(0.0)))
_G = 64
_CAP = 4
_GSLOTS = _G * _G * _CAP
_GPAD = _GSLOTS + _LANES
_OV = 256
_OVPAD = _OV + _LANES
_SENT = 1e18
_FULL = 8.9e17


def _nms_kernel_body(xys_hbm, order_hbm, sent_hbm,
                     keep_hbm, xyout_hbm, idxout_hbm,
                     xy_v, ord_v, alive_v, gx_v, gy_v, ox_v, oy_v,
                     xyout_v, idout_v, cnt_v, ovs):
    lane_iota = lax.iota(jnp.int32, _LANES)
    ones = jnp.broadcast_to(jnp.float32(1.0), (_LANES,))
    zeros = jnp.broadcast_to(jnp.float32(0.0), (_LANES,))

    @pl.when((lax.axis_index("c") == 0) & (lax.axis_index("s") == 0))
    def _work():
        pltpu.sync_copy(xys_hbm, xy_v)
        pltpu.sync_copy(order_hbm, ord_v)
        pltpu.sync_copy(sent_hbm, gx_v)
        pltpu.sync_copy(sent_hbm, gy_v)
        pltpu.sync_copy(sent_hbm.at[pl.ds(0, _OVPAD)], ox_v)
        pltpu.sync_copy(sent_hbm.at[pl.ds(0, _OVPAD)], oy_v)
        ovs[0] = jnp.int32(0)

        def _zc(c, carry):
            cnt_v[pl.ds(c * _LANES, _LANES)] = lane_iota * 0
            return carry

        lax.fori_loop(0, (_G * _G) // _LANES, _zc, 0)

        def _outer(cj, nselv):
            base = cj * _LANES
            sl = pl.ds(base, _LANES)
            ovi = ord_v[sl]
            xv = plsc.load_gather(xy_v, [ovi << 1])
            yv = plsc.load_gather(xy_v, [(ovi << 1) + 1])

            fxv = xv * 0.5
            fyv = yv * 0.5
            txv = fxv.astype(jnp.int32)
            tyv = fyv.astype(jnp.int32)
            ixv = txv - jnp.where(fxv < txv.astype(jnp.float32), 1, 0)
            iyv = tyv - jnp.where(fyv < tyv.astype(jnp.float32), 1, 0)
            hitv = zeros
            homebase = None
            for dyy in (-1, 0, 1):
                rowv = ((iyv + dyy) & (_G - 1)) << 6
                for dxx in (-1, 0, 1):
                    basev = (rowv + ((ixv + dxx) & (_G - 1))) << 2
                    if dxx == 0 and dyy == 0:
                        homebase = basev
                    for s in range(_CAP):
                        idxv = basev + s
                        gxs = plsc.load_gather(gx_v, [idxv])
                        gys = plsc.load_gather(gy_v, [idxv])
                        ddx = gxs - xv
                        ddy = gys - yv
                        d2 = ddx * ddx + ddy * ddy
                        hitv = jnp.where(d2 < _SUPPRESS_LT, ones, hitv)

            ovcnt = ovs[0]

            def _ovchunk(c, hv):
                o = pl.ds(c * _LANES, _LANES)
                oxc = ox_v[o]
                oyc = oy_v[o]
                for e in range(_LANES):
                    dxe = xv - oxc[e]
                    dye = yv - oyc[e]
                    d2e = dxe * dxe + dye * dye
                    hv = jnp.where(d2e < _SUPPRESS_LT, ones, hv)
                return hv

            hitv = lax.fori_loop(0, (ovcnt + 15) >> 4, _ovchunk, hitv)

            validv = jnp.where((base + lane_iota) < _N, ones, zeros)
            avp = (ones - hitv) * validv
            confv = zeros
            for o in range(1, 9):
                sh = (lane_iota + o) & (_LANES - 1)
                xr = xv.at[sh].get(mode="promise_in_bounds")
                yr = yv.at[sh].get(mode="promise_in_bounds")
                dxr = xv - xr
                dyr = yv - yr
                d2r = dxr * dxr + dyr * dyr
                confv = jnp.where(d2r < _SUPPRESS_LT, ones, confv)
            for s in (8, 4, 2, 1):
                confv = jnp.maximum(
                    confv, confv.at[lane_iota ^ s].get(mode="promise_in_bounds"))
            alive_v[sl] = avp

            @pl.when(confv[0] > 0.5)
            def _resolve():
                av = avp
                for l in range(_LANES):
                    dx = xv - xv[l]
                    dy = yv - yv[l]
                    d2 = dx * dx + dy * dy
                    gate = jnp.where(lane_iota > l, av[l], jnp.float32(0.0))
                    hitf = jnp.where(d2 < _SUPPRESS_LT, gate, zeros)
                    av = av * (ones - hitf)
                alive_v[sl] = av

            av = alive_v[sl]

            homecell = homebase >> 2
            cntv = plsc.load_gather(cnt_v, [homecell])
            izeros = lane_iota * 0
            iones = izeros + 1
            dupoff = izeros
            for o in range(1, _LANES):
                shl = (lane_iota - o) & (_LANES - 1)
                hb_b = homecell.at[shl].get(mode="promise_in_bounds")
                av_b = av.at[shl].get(mode="promise_in_bounds")
                sb = jnp.where(homecell == hb_b, av_b, jnp.float32(0.0))
                sb = jnp.where(lane_iota >= o, sb, jnp.float32(0.0))
                dupoff = dupoff + jnp.where(sb > 0.5, iones, izeros)
            slotv = cntv + dupoff
            okf = jnp.where(slotv < _CAP, av, zeros)
            plsc.store_scatter(gx_v, [homebase + slotv], xv, mask=okf > 0.5)
            plsc.store_scatter(gy_v, [homebase + slotv], yv, mask=okf > 0.5)
            plsc.addupdate_scatter(cnt_v, [homecell], iones, mask=av > 0.5)

            ovff = av - okf
            ovfs = ovff[0]
            for l in range(1, _LANES):
                ovfs = ovfs + ovff[l]

            @pl.when(ovfs > 0.5)
            def _ov_all():
                for l in range(_LANES):
                    @pl.when(ovff[l] > 0.5)
                    def _ov_ins(l=l):
                        ovc = ovs[0]

                        @pl.when(ovc < _OV)
                        def _ov_ins2():
                            ob = (ovc >> 4) << 4
                            olane = ovc - ob
                            osl = pl.ds(ob, _LANES)
                            ox_v[osl] = jnp.where(lane_iota == olane, xv[l],
                                                  ox_v[osl])
                            oy_v[osl] = jnp.where(lane_iota == olane, yv[l],
                                                  oy_v[osl])

                        ovs[0] = ovc + 1

            idout_v[sl] = jnp.where(av > 0.5, ovi, izeros - 1)
            oidx = (base + lane_iota) << 1
            plsc.store_scatter(xyout_v, [oidx], xv * av)
            plsc.store_scatter(xyout_v, [oidx + 1], yv * av)

            return nselv + av

        nselv = lax.fori_loop(0, _NCHUNK, _outer, zeros)
        nsel = nselv[0]
        for l in range(1, _LANES):
            nsel = nsel + nselv[l]

        need = jnp.maximum(jnp.float32(_MIN_LEN) - nsel, 0.0)

        @pl.when(need > 0.5)
        def _backfill():
            def _bf(c, run):
                base = c * _LANES
                sl = pl.ds(base, _LANES)
                av = alive_v[sl]
                newav = av
                for l in range(_LANES):
                    valid = (base + l) < _N
                    notk = valid & (av[l] < 0.5)
                    takef = jnp.where(notk & (run < need),
                                      jnp.float32(1.0), jnp.float32(0.0))
                    newav = newav + jnp.where(lane_iota == l, takef,
                                              jnp.float32(0.0))
                    run = run + jnp.where(notk, jnp.float32(1.0),
                                          jnp.float32(0.0))
                alive_v[sl] = newav
                ovi = ord_v[sl]
                xv = plsc.load_gather(xy_v, [ovi << 1])
                yv = plsc.load_gather(xy_v, [(ovi << 1) + 1])
                idout_v[sl] = jnp.where(newav > 0.5, ovi, (lane_iota * 0) - 1)
                oidx = (base + lane_iota) << 1
                plsc.store_scatter(xyout_v, [oidx], xv * newav)
                plsc.store_scatter(xyout_v, [oidx + 1], yv * newav)
                return run

            lax.fori_loop(0, _NCHUNK, _bf, jnp.float32(0.0))

        pltpu.sync_copy(alive_v.at[pl.ds(0, _N)], keep_hbm)
        pltpu.sync_copy(xyout_v.at[pl.ds(0, 2 * _N)], xyout_hbm)
        pltpu.sync_copy(idout_v.at[pl.ds(0, _N)], idxout_hbm)


@jax.jit
def _nms_run(xys_flat, order_pad, sent):
    fn = pl.kernel(
        _nms_kernel_body,
        out_type=(
            jax.ShapeDtypeStruct((_N,), jnp.float32),
            jax.ShapeDtypeStruct((2 * _N,), jnp.float32),
            jax.ShapeDtypeStruct((_N,), jnp.int32),
        ),
        mesh=plsc.VectorSubcoreMesh(core_axis_name="c", subcore_axis_name="s"),
        compiler_params=pltpu.CompilerParams(needs_layout_passes=False),
        scratch_types=[
            pltpu.VMEM((2 * _NPAD,), jnp.float32),
            pltpu.VMEM((_NPAD,), jnp.int32),
            pltpu.VMEM((_NPAD,), jnp.float32),
            pltpu.VMEM((_GPAD,), jnp.float32),
            pltpu.VMEM((_GPAD,), jnp.float32),
            pltpu.VMEM((_OVPAD,), jnp.float32),
            pltpu.VMEM((_OVPAD,), jnp.float32),
            pltpu.VMEM((2 * _NPAD,), jnp.float32),
            pltpu.VMEM((_NPAD,), jnp.int32),
            pltpu.VMEM((_G * _G,), jnp.int32),
            pltpu.SMEM((1,), jnp.int32),
        ],
    )
    return fn(xys_flat, order_pad, sent)


def kernel(xys, logits):
    order = jnp.argsort(-logits)
    order_pad = jnp.concatenate(
        [order, jnp.full((_NPAD - _N,), _N, dtype=jnp.int32)])
    xys_flat = jnp.concatenate(
        [jnp.reshape(xys, (-1,)),
         jnp.full((2 * (_NPAD - _N),), 1e9, dtype=jnp.float32)])
    sent = jnp.full((_GPAD,), _SENT, dtype=jnp.float32)
    keep_f, xy_flat, selected_idcs = _nms_run(xys_flat, order_pad, sent)
    keep_final = keep_f > 0.5
    selected_xys = jnp.reshape(xy_flat, (_N, 2))
    return selected_idcs, selected_xys, keep_final
